# Initial kernel scaffold; baseline (speedup 1.0000x reference)
import functools

import jax
import jax.numpy as jnp
from jax import lax
from jax.experimental import pallas as pl
from jax.experimental.pallas import tpu as pltpu
from jax.experimental.pallas import tpu_sc as plsc

N_NODES = 10000
N_EDGES = 320000

NC = 2
NS = 16
NW = NC * NS
EDGES_PER_TILE = N_EDGES // NW
CHUNK = 80
NCHUNK = EDGES_PER_TILE // CHUNK
ROWS_PER_TILE = N_NODES // NS
DEG_W = 8

_mesh = plsc.VectorSubcoreMesh(core_axis_name="c", subcore_axis_name="s",
                               num_cores=NC, num_subcores=NS)


def _wid(c, s):
    return s * NC + c


def _sc_degree(dst3, ones_h, zeros_h, deg_part, idx_v, ones_v, acc_sh, sem):
    c = lax.axis_index("c")
    s = lax.axis_index("s")
    w = _wid(c, s)
    pltpu.sync_copy(zeros_h.at[pl.ds(s * ROWS_PER_TILE, ROWS_PER_TILE)],
                    acc_sh.at[pl.ds(s * ROWS_PER_TILE, ROWS_PER_TILE)])
    pltpu.sync_copy(ones_h, ones_v)
    pltpu.sync_copy(dst3.at[w], idx_v)
    plsc.subcore_barrier()

    def body(j, carry):
        pltpu.sync_copy(ones_v, acc_sh.at[idx_v.at[j]], add=True)
        return carry

    lax.fori_loop(0, NCHUNK, body, 0)
    plsc.subcore_barrier()
    pltpu.sync_copy(acc_sh.at[pl.ds(s * ROWS_PER_TILE, ROWS_PER_TILE)],
                    deg_part.at[c, pl.ds(s * ROWS_PER_TILE, ROWS_PER_TILE)])


def _degree(dst3, ones_h, zeros_h):
    f = pl.kernel(
        _sc_degree,
        out_type=jax.ShapeDtypeStruct((NC, N_NODES, DEG_W), jnp.float32),
        mesh=_mesh,
        scratch_types=[
            pltpu.VMEM((NCHUNK, CHUNK), jnp.int32),
            pltpu.VMEM((CHUNK, DEG_W), jnp.float32),
            pltpu.MemoryRef((N_NODES, DEG_W), jnp.float32,
                            memory_space=pltpu.VMEM_SHARED),
            pltpu.SemaphoreType.DMA,
        ],
    )
    return f(dst3, ones_h, zeros_h)


def _sc_aggregate(src3, dst3, xws, zeros_h, part,
                  sidx_v, didx_v, rows_v, acc_sh, sem):
    c = lax.axis_index("c")
    s = lax.axis_index("s")
    w = _wid(c, s)
    pltpu.sync_copy(zeros_h.at[pl.ds(s * ROWS_PER_TILE, ROWS_PER_TILE)],
                    acc_sh.at[pl.ds(s * ROWS_PER_TILE, ROWS_PER_TILE)])
    pltpu.sync_copy(src3.at[w], sidx_v)
    pltpu.sync_copy(dst3.at[w], didx_v)
    plsc.subcore_barrier()

    def body(j, carry):
        pltpu.async_copy(xws.at[sidx_v.at[j]], rows_v, sem).wait()
        pltpu.sync_copy(rows_v, acc_sh.at[didx_v.at[j]], add=True)
        return carry

    lax.fori_loop(0, NCHUNK, body, 0)
    plsc.subcore_barrier()
    pltpu.sync_copy(acc_sh.at[pl.ds(s * ROWS_PER_TILE, ROWS_PER_TILE)],
                    part.at[c, pl.ds(s * ROWS_PER_TILE, ROWS_PER_TILE)])


def _aggregate(src3, dst3, xws, zeros_h):
    d = xws.shape[1]
    f = pl.kernel(
        _sc_aggregate,
        out_type=jax.ShapeDtypeStruct((NC, N_NODES, d), jnp.float32),
        mesh=_mesh,
        scratch_types=[
            pltpu.VMEM((NCHUNK, CHUNK), jnp.int32),
            pltpu.VMEM((NCHUNK, CHUNK), jnp.int32),
            pltpu.VMEM((CHUNK, d), jnp.float32),
            pltpu.MemoryRef((N_NODES, d), jnp.float32,
                            memory_space=pltpu.VMEM_SHARED),
            pltpu.SemaphoreType.DMA,
        ],
    )
    return f(src3, dst3, xws, zeros_h)


BR = 500


def _tc_stage1(deg_ref, x_ref, w_ref, dinv_ref, xws_ref):
    deg = deg_ref[0, :, 0:1] + deg_ref[1, :, 0:1] + 1.0
    dinv = lax.rsqrt(deg)
    dinv_ref[...] = dinv
    xw = jnp.dot(x_ref[...], w_ref[...],
                 preferred_element_type=jnp.float32,
                 precision=lax.Precision.HIGHEST)
    xws_ref[...] = xw * dinv


def _stage1(deg_part, x, W1):
    d_in, d = W1.shape
    grid = (N_NODES // BR,)
    return pl.pallas_call(
        _tc_stage1,
        grid=grid,
        in_specs=[
            pl.BlockSpec((NC, BR, DEG_W), lambda i: (0, i, 0)),
            pl.BlockSpec((BR, d_in), lambda i: (i, 0)),
            pl.BlockSpec((d_in, d), lambda i: (0, 0)),
        ],
        out_specs=[
            pl.BlockSpec((BR, 1), lambda i: (i, 0)),
            pl.BlockSpec((BR, d), lambda i: (i, 0)),
        ],
        out_shape=[
            jax.ShapeDtypeStruct((N_NODES, 1), jnp.float32),
            jax.ShapeDtypeStruct((N_NODES, d), jnp.float32),
        ],
    )(deg_part, x, W1)


def _tc_stage_mid(part_ref, xws_ref, dinv_ref, b_ref, w_ref, out_ref):
    dinv = dinv_ref[...]
    tot = part_ref[0] + part_ref[1] + xws_ref[...]
    h = dinv * tot + b_ref[...]
    h = jnp.where(h > 0, h, jnp.exp(h) - 1.0)
    hw = jnp.dot(h, w_ref[...],
                 preferred_element_type=jnp.float32,
                 precision=lax.Precision.HIGHEST)
    out_ref[...] = hw * dinv


def _stage_mid(part, xws, dinv, b, W):
    d_in, d = W.shape
    grid = (N_NODES // BR,)
    return pl.pallas_call(
        _tc_stage_mid,
        grid=grid,
        in_specs=[
            pl.BlockSpec((NC, BR, d_in), lambda i: (0, i, 0)),
            pl.BlockSpec((BR, d_in), lambda i: (i, 0)),
            pl.BlockSpec((BR, 1), lambda i: (i, 0)),
            pl.BlockSpec((1, d_in), lambda i: (0, 0)),
            pl.BlockSpec((d_in, d), lambda i: (0, 0)),
        ],
        out_specs=pl.BlockSpec((BR, d), lambda i: (i, 0)),
        out_shape=jax.ShapeDtypeStruct((N_NODES, d), jnp.float32),
    )(part, xws, dinv, b.reshape(1, d_in), W)


def _tc_stage_out(part_ref, xws_ref, dinv_ref, b_ref, out_ref):
    tot = part_ref[0] + part_ref[1] + xws_ref[...]
    out_ref[...] = dinv_ref[...] * tot + b_ref[...]


def _stage_out(part, xws, dinv, b):
    d = xws.shape[1]
    grid = (N_NODES // BR,)
    return pl.pallas_call(
        _tc_stage_out,
        grid=grid,
        in_specs=[
            pl.BlockSpec((NC, BR, d), lambda i: (0, i, 0)),
            pl.BlockSpec((BR, d), lambda i: (i, 0)),
            pl.BlockSpec((BR, 1), lambda i: (i, 0)),
            pl.BlockSpec((1, d), lambda i: (0, 0)),
        ],
        out_specs=pl.BlockSpec((BR, d), lambda i: (i, 0)),
        out_shape=jax.ShapeDtypeStruct((N_NODES, d), jnp.float32),
    )(part, xws, dinv, b.reshape(1, d))


def kernel(x, edge_index, W1, b1, W2, b2, W3, b3):
    src = edge_index[0].astype(jnp.int32)
    dst = edge_index[1].astype(jnp.int32)
    src3 = src.reshape(NW, NCHUNK, CHUNK)
    dst3 = dst.reshape(NW, NCHUNK, CHUNK)

    zeros32 = jnp.zeros((N_NODES, 32), jnp.float32)
    zeros16 = jnp.zeros((N_NODES, 16), jnp.float32)
    zeros_deg = jnp.zeros((N_NODES, DEG_W), jnp.float32)
    ones_h = jnp.ones((CHUNK, DEG_W), jnp.float32)

    deg_part = _degree(dst3, ones_h, zeros_deg)
    dinv, xws1 = _stage1(deg_part, x, W1)

    part1 = _aggregate(src3, dst3, xws1, zeros32)
    xws2 = _stage_mid(part1, xws1, dinv, b1, W2)

    part2 = _aggregate(src3, dst3, xws2, zeros32)
    xws3 = _stage_mid(part2, xws2, dinv, b2, W3)

    part3 = _aggregate(src3, dst3, xws3, zeros16)
    out = _stage_out(part3, xws3, dinv, b3)
    return out


# speedup vs baseline: 23.5354x; 23.5354x over previous
import functools

import jax
import jax.numpy as jnp
from jax import lax
from jax.experimental import pallas as pl
from jax.experimental.pallas import tpu as pltpu
from jax.experimental.pallas import tpu_sc as plsc

N_NODES = 10000
NPAD = 10240
N_EDGES = 320000

NC = 2
NS = 16
NW = NC * NS
EDGES_PER_TILE = N_EDGES // NW
CHUNK = 80
NCHUNK = EDGES_PER_TILE // CHUNK
ROWS_PER_TILE = NPAD // NS
DEG_W = 8

_mesh = plsc.VectorSubcoreMesh(core_axis_name="c", subcore_axis_name="s",
                               num_cores=NC, num_subcores=NS)
_sc_params = pltpu.CompilerParams(use_tc_tiling_on_sc=False)


def _wid(c, s):
    return s * NC + c


def _sc_degree(dst3, ones_h, zeros_h, deg_part, idx_v, ones_v, acc_sh, sem):
    c = lax.axis_index("c")
    s = lax.axis_index("s")
    w = _wid(c, s)
    pltpu.sync_copy(zeros_h.at[pl.ds(s * ROWS_PER_TILE, ROWS_PER_TILE)],
                    acc_sh.at[pl.ds(s * ROWS_PER_TILE, ROWS_PER_TILE)])
    pltpu.sync_copy(ones_h, ones_v)
    pltpu.sync_copy(dst3.at[w], idx_v)
    plsc.subcore_barrier()

    def body(j, carry):
        pltpu.sync_copy(ones_v, acc_sh.at[idx_v.at[j]], add=True)
        return carry

    lax.fori_loop(0, NCHUNK, body, 0)
    plsc.subcore_barrier()
    pltpu.sync_copy(acc_sh.at[pl.ds(s * ROWS_PER_TILE, ROWS_PER_TILE)],
                    deg_part.at[c, pl.ds(s * ROWS_PER_TILE, ROWS_PER_TILE)])


def _degree(dst3, ones_h, zeros_h):
    f = pl.kernel(
        _sc_degree,
        out_type=jax.ShapeDtypeStruct((NC, NPAD, DEG_W), jnp.float32),
        mesh=_mesh,
        compiler_params=_sc_params,
        scratch_types=[
            pltpu.VMEM((NCHUNK, CHUNK), jnp.int32),
            pltpu.VMEM((CHUNK, DEG_W), jnp.float32),
            pltpu.VMEM_SHARED((NPAD, DEG_W), jnp.float32),
            pltpu.SemaphoreType.DMA,
        ],
    )
    return f(dst3, ones_h, zeros_h)


def _sc_aggregate(src3, dst3, xws, zeros_h, part,
                  sidx_v, didx_v, rows_v, acc_sh, sem):
    c = lax.axis_index("c")
    s = lax.axis_index("s")
    w = _wid(c, s)
    pltpu.sync_copy(zeros_h.at[pl.ds(s * ROWS_PER_TILE, ROWS_PER_TILE)],
                    acc_sh.at[pl.ds(s * ROWS_PER_TILE, ROWS_PER_TILE)])
    pltpu.sync_copy(src3.at[w], sidx_v)
    pltpu.sync_copy(dst3.at[w], didx_v)
    plsc.subcore_barrier()

    def body(j, carry):
        pltpu.async_copy(xws.at[sidx_v.at[j]], rows_v, sem).wait()
        pltpu.sync_copy(rows_v, acc_sh.at[didx_v.at[j]], add=True)
        return carry

    lax.fori_loop(0, NCHUNK, body, 0)
    plsc.subcore_barrier()
    pltpu.sync_copy(acc_sh.at[pl.ds(s * ROWS_PER_TILE, ROWS_PER_TILE)],
                    part.at[c, pl.ds(s * ROWS_PER_TILE, ROWS_PER_TILE)])


def _aggregate(src3, dst3, xws, zeros_h):
    d = xws.shape[1]
    f = pl.kernel(
        _sc_aggregate,
        out_type=jax.ShapeDtypeStruct((NC, NPAD, d), jnp.float32),
        mesh=_mesh,
        compiler_params=_sc_params,
        scratch_types=[
            pltpu.VMEM((NCHUNK, CHUNK), jnp.int32),
            pltpu.VMEM((NCHUNK, CHUNK), jnp.int32),
            pltpu.VMEM((CHUNK, d), jnp.float32),
            pltpu.VMEM_SHARED((NPAD, d), jnp.float32),
            pltpu.SemaphoreType.DMA,
        ],
    )
    return f(src3, dst3, xws, zeros_h)


BR = 1000


def _tc_stage1(deg_ref, x_ref, w_ref, dinv_ref, xws_ref):
    deg = deg_ref[0, :, 0:1] + deg_ref[1, :, 0:1] + 1.0
    dinv = lax.rsqrt(deg)
    dinv_ref[...] = dinv
    xw = jnp.dot(x_ref[...], w_ref[...],
                 preferred_element_type=jnp.float32,
                 precision=lax.Precision.HIGHEST)
    xws_ref[...] = xw * dinv


def _stage1(deg_part, x, W1):
    d_in, d = W1.shape
    grid = (N_NODES // BR,)
    return pl.pallas_call(
        _tc_stage1,
        grid=grid,
        in_specs=[
            pl.BlockSpec((NC, BR, DEG_W), lambda i: (0, i, 0)),
            pl.BlockSpec((BR, d_in), lambda i: (i, 0)),
            pl.BlockSpec((d_in, d), lambda i: (0, 0)),
        ],
        out_specs=[
            pl.BlockSpec((BR, 1), lambda i: (i, 0)),
            pl.BlockSpec((BR, d), lambda i: (i, 0)),
        ],
        out_shape=[
            jax.ShapeDtypeStruct((N_NODES, 1), jnp.float32),
            jax.ShapeDtypeStruct((N_NODES, d), jnp.float32),
        ],
    )(deg_part, x, W1)


def _tc_stage_mid(part_ref, xws_ref, dinv_ref, b_ref, w_ref, out_ref):
    dinv = dinv_ref[...]
    tot = part_ref[0] + part_ref[1] + xws_ref[...]
    h = dinv * tot + b_ref[...]
    h = jnp.where(h > 0, h, jnp.exp(h) - 1.0)
    hw = jnp.dot(h, w_ref[...],
                 preferred_element_type=jnp.float32,
                 precision=lax.Precision.HIGHEST)
    out_ref[...] = hw * dinv


def _stage_mid(part, xws, dinv, b, W):
    d_in, d = W.shape
    grid = (N_NODES // BR,)
    return pl.pallas_call(
        _tc_stage_mid,
        grid=grid,
        in_specs=[
            pl.BlockSpec((NC, BR, d_in), lambda i: (0, i, 0)),
            pl.BlockSpec((BR, d_in), lambda i: (i, 0)),
            pl.BlockSpec((BR, 1), lambda i: (i, 0)),
            pl.BlockSpec((1, d_in), lambda i: (0, 0)),
            pl.BlockSpec((d_in, d), lambda i: (0, 0)),
        ],
        out_specs=pl.BlockSpec((BR, d), lambda i: (i, 0)),
        out_shape=jax.ShapeDtypeStruct((N_NODES, d), jnp.float32),
    )(part, xws, dinv, b.reshape(1, d_in), W)


def _tc_stage_out(part_ref, xws_ref, dinv_ref, b_ref, out_ref):
    tot = part_ref[0] + part_ref[1] + xws_ref[...]
    out_ref[...] = dinv_ref[...] * tot + b_ref[...]


def _stage_out(part, xws, dinv, b):
    d = xws.shape[1]
    grid = (N_NODES // BR,)
    return pl.pallas_call(
        _tc_stage_out,
        grid=grid,
        in_specs=[
            pl.BlockSpec((NC, BR, d), lambda i: (0, i, 0)),
            pl.BlockSpec((BR, d), lambda i: (i, 0)),
            pl.BlockSpec((BR, 1), lambda i: (i, 0)),
            pl.BlockSpec((1, d), lambda i: (0, 0)),
        ],
        out_specs=pl.BlockSpec((BR, d), lambda i: (i, 0)),
        out_shape=jax.ShapeDtypeStruct((N_NODES, d), jnp.float32),
    )(part, xws, dinv, b.reshape(1, d))


def kernel(x, edge_index, W1, b1, W2, b2, W3, b3):
    src = edge_index[0].astype(jnp.int32)
    dst = edge_index[1].astype(jnp.int32)
    src3 = src.reshape(NW, NCHUNK, CHUNK)
    dst3 = dst.reshape(NW, NCHUNK, CHUNK)

    zeros32 = jnp.zeros((NPAD, 32), jnp.float32)
    zeros16 = jnp.zeros((NPAD, 16), jnp.float32)
    zeros_deg = jnp.zeros((NPAD, DEG_W), jnp.float32)
    ones_h = jnp.ones((CHUNK, DEG_W), jnp.float32)

    deg_part = _degree(dst3, ones_h, zeros_deg)
    dinv, xws1 = _stage1(deg_part, x, W1)

    part1 = _aggregate(src3, dst3, xws1, zeros32)
    xws2 = _stage_mid(part1, xws1, dinv, b1, W2)

    part2 = _aggregate(src3, dst3, xws2, zeros32)
    xws3 = _stage_mid(part2, xws2, dinv, b2, W3)

    part3 = _aggregate(src3, dst3, xws3, zeros16)
    out = _stage_out(part3, xws3, dinv, b3)
    return out


# speedup vs baseline: 41.4686x; 1.7620x over previous
import functools

import jax
import jax.numpy as jnp
from jax import lax
from jax.experimental import pallas as pl
from jax.experimental.pallas import tpu as pltpu
from jax.experimental.pallas import tpu_sc as plsc

N_NODES = 10000
NPAD = 10240
N_EDGES = 320000

NC = 2
NS = 16
NW = NC * NS
EDGES_PER_TILE = N_EDGES // NW
CHUNK = 80
NCHUNK = EDGES_PER_TILE // CHUNK
ROWS_PER_TILE = NPAD // NS
DEG_W = 8

_mesh = plsc.VectorSubcoreMesh(core_axis_name="c", subcore_axis_name="s",
                               num_cores=NC, num_subcores=NS)
_sc_params = pltpu.CompilerParams(use_tc_tiling_on_sc=False)


def _wid(c, s):
    return s * NC + c


def _sc_degree(dst3, ones_h, zeros_h, deg_part, idx_v, ones_v, acc_sh, sem):
    c = lax.axis_index("c")
    s = lax.axis_index("s")
    w = _wid(c, s)
    pltpu.sync_copy(zeros_h.at[pl.ds(s * ROWS_PER_TILE, ROWS_PER_TILE)],
                    acc_sh.at[pl.ds(s * ROWS_PER_TILE, ROWS_PER_TILE)])
    pltpu.sync_copy(ones_h, ones_v)
    pltpu.sync_copy(dst3.at[w], idx_v)
    plsc.subcore_barrier()

    def body(j, carry):
        pltpu.sync_copy(ones_v, acc_sh.at[idx_v.at[j]], add=True)
        return carry

    lax.fori_loop(0, NCHUNK, body, 0)
    plsc.subcore_barrier()
    pltpu.sync_copy(acc_sh.at[pl.ds(s * ROWS_PER_TILE, ROWS_PER_TILE)],
                    deg_part.at[c, pl.ds(s * ROWS_PER_TILE, ROWS_PER_TILE)])


def _degree(dst3, ones_h, zeros_h):
    f = pl.kernel(
        _sc_degree,
        out_type=jax.ShapeDtypeStruct((NC, NPAD, DEG_W), jnp.float32),
        mesh=_mesh,
        compiler_params=_sc_params,
        scratch_types=[
            pltpu.VMEM((NCHUNK, CHUNK), jnp.int32),
            pltpu.VMEM((CHUNK, DEG_W), jnp.float32),
            pltpu.VMEM_SHARED((NPAD, DEG_W), jnp.float32),
            pltpu.SemaphoreType.DMA,
        ],
    )
    return f(dst3, ones_h, zeros_h)


NBUF = 4
LOOKAHEAD = 2


def _sc_aggregate(src3, dst3, xws, zeros_h, part,
                  sidx_v, didx_v, rows_v, acc_sh, gsem, ssem):
    c = lax.axis_index("c")
    s = lax.axis_index("s")
    w = _wid(c, s)
    pltpu.sync_copy(zeros_h.at[pl.ds(s * ROWS_PER_TILE, ROWS_PER_TILE)],
                    acc_sh.at[pl.ds(s * ROWS_PER_TILE, ROWS_PER_TILE)])
    pltpu.sync_copy(src3.at[w], sidx_v)
    pltpu.sync_copy(dst3.at[w], didx_v)
    plsc.subcore_barrier()

    def gather(j, b):
        pltpu.async_copy(xws.at[sidx_v.at[j]], rows_v.at[b], gsem.at[b])

    def scatter(j, b):
        pltpu.async_copy(rows_v.at[b], acc_sh.at[didx_v.at[j]], ssem.at[b],
                         add=True)

    for j in range(LOOKAHEAD):
        gather(j, j % NBUF)

    def body(j, carry):
        bg = (j + LOOKAHEAD) % NBUF

        @pl.when(j + LOOKAHEAD < NCHUNK)
        def _():
            @pl.when(j + LOOKAHEAD >= NBUF)
            def _():
                pltpu.make_async_copy(rows_v.at[bg],
                                      acc_sh.at[didx_v.at[0]],
                                      ssem.at[bg]).wait()
            gather(j + LOOKAHEAD, bg)

        b = j % NBUF
        pltpu.make_async_copy(xws.at[sidx_v.at[j]], rows_v.at[b],
                              gsem.at[b]).wait()
        scatter(j, b)
        return carry

    lax.fori_loop(0, NCHUNK, body, 0)
    for b in range(NBUF):
        pltpu.make_async_copy(rows_v.at[b], acc_sh.at[didx_v.at[0]],
                              ssem.at[b]).wait()
    plsc.subcore_barrier()
    pltpu.sync_copy(acc_sh.at[pl.ds(s * ROWS_PER_TILE, ROWS_PER_TILE)],
                    part.at[c, pl.ds(s * ROWS_PER_TILE, ROWS_PER_TILE)])


def _aggregate(src3, dst3, xws, zeros_h):
    d = xws.shape[1]
    f = pl.kernel(
        _sc_aggregate,
        out_type=jax.ShapeDtypeStruct((NC, NPAD, d), jnp.float32),
        mesh=_mesh,
        compiler_params=_sc_params,
        scratch_types=[
            pltpu.VMEM((NCHUNK, CHUNK), jnp.int32),
            pltpu.VMEM((NCHUNK, CHUNK), jnp.int32),
            pltpu.VMEM((NBUF, CHUNK, d), jnp.float32),
            pltpu.VMEM_SHARED((NPAD, d), jnp.float32),
            pltpu.SemaphoreType.DMA((NBUF,)),
            pltpu.SemaphoreType.DMA((NBUF,)),
        ],
    )
    return f(src3, dst3, xws, zeros_h)


BR = 1000


def _tc_stage1(deg_ref, x_ref, w_ref, dinv_ref, xws_ref):
    deg = deg_ref[0, :, 0:1] + deg_ref[1, :, 0:1] + 1.0
    dinv = lax.rsqrt(deg)
    dinv_ref[...] = dinv
    xw = jnp.dot(x_ref[...], w_ref[...],
                 preferred_element_type=jnp.float32,
                 precision=lax.Precision.HIGHEST)
    xws_ref[...] = xw * dinv


def _stage1(deg_part, x, W1):
    d_in, d = W1.shape
    grid = (N_NODES // BR,)
    return pl.pallas_call(
        _tc_stage1,
        grid=grid,
        in_specs=[
            pl.BlockSpec((NC, BR, DEG_W), lambda i: (0, i, 0)),
            pl.BlockSpec((BR, d_in), lambda i: (i, 0)),
            pl.BlockSpec((d_in, d), lambda i: (0, 0)),
        ],
        out_specs=[
            pl.BlockSpec((BR, 1), lambda i: (i, 0)),
            pl.BlockSpec((BR, d), lambda i: (i, 0)),
        ],
        out_shape=[
            jax.ShapeDtypeStruct((N_NODES, 1), jnp.float32),
            jax.ShapeDtypeStruct((N_NODES, d), jnp.float32),
        ],
    )(deg_part, x, W1)


def _tc_stage_mid(part_ref, xws_ref, dinv_ref, b_ref, w_ref, out_ref):
    dinv = dinv_ref[...]
    tot = part_ref[0] + part_ref[1] + xws_ref[...]
    h = dinv * tot + b_ref[...]
    h = jnp.where(h > 0, h, jnp.exp(h) - 1.0)
    hw = jnp.dot(h, w_ref[...],
                 preferred_element_type=jnp.float32,
                 precision=lax.Precision.HIGHEST)
    out_ref[...] = hw * dinv


def _stage_mid(part, xws, dinv, b, W):
    d_in, d = W.shape
    grid = (N_NODES // BR,)
    return pl.pallas_call(
        _tc_stage_mid,
        grid=grid,
        in_specs=[
            pl.BlockSpec((NC, BR, d_in), lambda i: (0, i, 0)),
            pl.BlockSpec((BR, d_in), lambda i: (i, 0)),
            pl.BlockSpec((BR, 1), lambda i: (i, 0)),
            pl.BlockSpec((1, d_in), lambda i: (0, 0)),
            pl.BlockSpec((d_in, d), lambda i: (0, 0)),
        ],
        out_specs=pl.BlockSpec((BR, d), lambda i: (i, 0)),
        out_shape=jax.ShapeDtypeStruct((N_NODES, d), jnp.float32),
    )(part, xws, dinv, b.reshape(1, d_in), W)


def _tc_stage_out(part_ref, xws_ref, dinv_ref, b_ref, out_ref):
    tot = part_ref[0] + part_ref[1] + xws_ref[...]
    out_ref[...] = dinv_ref[...] * tot + b_ref[...]


def _stage_out(part, xws, dinv, b):
    d = xws.shape[1]
    grid = (N_NODES // BR,)
    return pl.pallas_call(
        _tc_stage_out,
        grid=grid,
        in_specs=[
            pl.BlockSpec((NC, BR, d), lambda i: (0, i, 0)),
            pl.BlockSpec((BR, d), lambda i: (i, 0)),
            pl.BlockSpec((BR, 1), lambda i: (i, 0)),
            pl.BlockSpec((1, d), lambda i: (0, 0)),
        ],
        out_specs=pl.BlockSpec((BR, d), lambda i: (i, 0)),
        out_shape=jax.ShapeDtypeStruct((N_NODES, d), jnp.float32),
    )(part, xws, dinv, b.reshape(1, d))


def kernel(x, edge_index, W1, b1, W2, b2, W3, b3):
    src = edge_index[0].astype(jnp.int32)
    dst = edge_index[1].astype(jnp.int32)
    src3 = src.reshape(NW, NCHUNK, CHUNK)
    dst3 = dst.reshape(NW, NCHUNK, CHUNK)

    zeros32 = jnp.zeros((NPAD, 32), jnp.float32)
    zeros16 = jnp.zeros((NPAD, 16), jnp.float32)
    zeros_deg = jnp.zeros((NPAD, DEG_W), jnp.float32)
    ones_h = jnp.ones((CHUNK, DEG_W), jnp.float32)

    deg_part = _degree(dst3, ones_h, zeros_deg)
    dinv, xws1 = _stage1(deg_part, x, W1)

    part1 = _aggregate(src3, dst3, xws1, zeros32)
    xws2 = _stage_mid(part1, xws1, dinv, b1, W2)

    part2 = _aggregate(src3, dst3, xws2, zeros32)
    xws3 = _stage_mid(part2, xws2, dinv, b2, W3)

    part3 = _aggregate(src3, dst3, xws3, zeros16)
    out = _stage_out(part3, xws3, dinv, b3)
    return out


# speedup vs baseline: 60.0818x; 1.4488x over previous
import jax
import jax.numpy as jnp
from jax import lax
from jax.experimental import pallas as pl
from jax.experimental.pallas import tpu as pltpu
from jax.experimental.pallas import tpu_sc as plsc

N_NODES = 10000
NPAD = 10240
N_EDGES = 320000
D = 32
PK = 128 // D
NPK = NPAD // PK

NC = 2
NS = 16
NW = NC * NS
E_PAD = 327680
EDGES_PER_TILE = E_PAD // NW
CHUNK = 128
NCHUNK = EDGES_PER_TILE // CHUNK
ROWS_PER_TILE = NPAD // NS
PROWS_PER_TILE = ROWS_PER_TILE // PK
DEG_W = 8
NBUF = 4
LOOKAHEAD = 2

_mesh = plsc.VectorSubcoreMesh(core_axis_name="c", subcore_axis_name="s",
                               num_cores=NC, num_subcores=NS)
_sc_params = pltpu.CompilerParams(use_tc_tiling_on_sc=False,
                                  needs_layout_passes=False)


def _wid(c, s):
    return s * NC + c


def _sc_degree(dst3, ones_h, zeros_h, deg_p,
               idx_v, ones_v, slab_v, pack_v, acc_sh, sem):
    c = lax.axis_index("c")
    s = lax.axis_index("s")
    w = _wid(c, s)
    pltpu.sync_copy(zeros_h.at[pl.ds(s * ROWS_PER_TILE, ROWS_PER_TILE)],
                    acc_sh.at[pl.ds(s * ROWS_PER_TILE, ROWS_PER_TILE)])
    pltpu.sync_copy(ones_h, ones_v)
    pltpu.sync_copy(dst3.at[w], idx_v)
    plsc.subcore_barrier()

    def body(j, carry):
        @pl.when(j >= NBUF)
        def _():
            pltpu.make_async_copy(ones_v, acc_sh.at[idx_v.at[0]],
                                  sem.at[j % NBUF]).wait()
        pltpu.async_copy(ones_v, acc_sh.at[idx_v.at[j]], sem.at[j % NBUF],
                         add=True)
        return carry

    lax.fori_loop(0, NCHUNK, body, 0)
    for b in range(NBUF):
        pltpu.make_async_copy(ones_v, acc_sh.at[idx_v.at[0]],
                              sem.at[b]).wait()
    plsc.subcore_barrier()

    pltpu.sync_copy(acc_sh.at[pl.ds(s * ROWS_PER_TILE, ROWS_PER_TILE)],
                    slab_v)

    def pack_row(i, carry):
        for half in range(8):
            node = PK * i + half // 2
            idx = jnp.full((16,), node, jnp.int32)
            col = jnp.zeros((16,), jnp.int32)
            v = plsc.load_gather(slab_v, [idx, col])
            pack_v[i, pl.ds(16 * half, 16)] = v
        return carry

    lax.fori_loop(0, PROWS_PER_TILE, pack_row, 0)
    pltpu.sync_copy(pack_v,
                    deg_p.at[c, pl.ds(s * PROWS_PER_TILE, PROWS_PER_TILE)])


def _degree(dst3, ones_h, zeros_h):
    f = pl.kernel(
        _sc_degree,
        out_type=jax.ShapeDtypeStruct((NC, NPK, 128), jnp.float32),
        mesh=_mesh,
        compiler_params=_sc_params,
        scratch_types=[
            pltpu.VMEM((NCHUNK, CHUNK), jnp.int32),
            pltpu.VMEM((CHUNK, DEG_W), jnp.float32),
            pltpu.VMEM((ROWS_PER_TILE, DEG_W), jnp.float32),
            pltpu.VMEM((PROWS_PER_TILE, 128), jnp.float32),
            pltpu.VMEM_SHARED((NPAD, DEG_W), jnp.float32),
            pltpu.SemaphoreType.DMA((NBUF,)),
        ],
    )
    return f(dst3, ones_h, zeros_h)


def _sc_aggregate(src3, dst3, xws, zeros_h, part,
                  sidx_v, didx_v, rows_v, acc_sh, gsem, ssem):
    c = lax.axis_index("c")
    s = lax.axis_index("s")
    w = _wid(c, s)
    pltpu.sync_copy(zeros_h.at[pl.ds(s * ROWS_PER_TILE, ROWS_PER_TILE)],
                    acc_sh.at[pl.ds(s * ROWS_PER_TILE, ROWS_PER_TILE)])
    pltpu.sync_copy(src3.at[w], sidx_v)
    pltpu.sync_copy(dst3.at[w], didx_v)
    plsc.subcore_barrier()

    def gather(j, b):
        pltpu.async_copy(xws.at[sidx_v.at[j]], rows_v.at[b], gsem.at[b])

    def scatter(j, b):
        pltpu.async_copy(rows_v.at[b], acc_sh.at[didx_v.at[j]], ssem.at[b],
                         add=True)

    for j in range(LOOKAHEAD):
        gather(j, j % NBUF)

    def body(j, carry):
        bg = (j + LOOKAHEAD) % NBUF

        @pl.when(j + LOOKAHEAD < NCHUNK)
        def _():
            @pl.when(j + LOOKAHEAD >= NBUF)
            def _():
                pltpu.make_async_copy(rows_v.at[bg],
                                      acc_sh.at[didx_v.at[0]],
                                      ssem.at[bg]).wait()
            gather(j + LOOKAHEAD, bg)

        b = j % NBUF
        pltpu.make_async_copy(xws.at[sidx_v.at[j]], rows_v.at[b],
                              gsem.at[b]).wait()
        scatter(j, b)
        return carry

    lax.fori_loop(0, NCHUNK, body, 0)
    for b in range(NBUF):
        pltpu.make_async_copy(rows_v.at[b], acc_sh.at[didx_v.at[0]],
                              ssem.at[b]).wait()
    plsc.subcore_barrier()
    pltpu.sync_copy(acc_sh.at[pl.ds(s * ROWS_PER_TILE, ROWS_PER_TILE)],
                    part.at[c, pl.ds(s * ROWS_PER_TILE, ROWS_PER_TILE)])


def _aggregate(src3, dst3, xws, zeros_h):
    f = pl.kernel(
        _sc_aggregate,
        out_type=jax.ShapeDtypeStruct((NC, NPAD, D), jnp.float32),
        mesh=_mesh,
        compiler_params=_sc_params,
        scratch_types=[
            pltpu.VMEM((NCHUNK, CHUNK), jnp.int32),
            pltpu.VMEM((NCHUNK, CHUNK), jnp.int32),
            pltpu.VMEM((NBUF, CHUNK, D), jnp.float32),
            pltpu.VMEM_SHARED((NPAD, D), jnp.float32),
            pltpu.SemaphoreType.DMA((NBUF,)),
            pltpu.SemaphoreType.DMA((NBUF,)),
        ],
    )
    return f(src3, dst3, xws, zeros_h)


BRP = 320


def _tc_stage1(deg_ref, xp_ref, w_ref, dinv_ref, xws_ref):
    dinv = lax.rsqrt(deg_ref[0] + deg_ref[1] + 1.0)
    dinv_ref[...] = dinv
    xw = jnp.dot(xp_ref[...], w_ref[...],
                 preferred_element_type=jnp.float32,
                 precision=lax.Precision.HIGHEST)
    xws_ref[...] = xw * dinv


def _stage1(deg_p, xp, Wbd1):
    grid = (NPK // BRP,)
    return pl.pallas_call(
        _tc_stage1,
        grid=grid,
        in_specs=[
            pl.BlockSpec((NC, BRP, 128), lambda i: (0, i, 0)),
            pl.BlockSpec((BRP, PK * 128), lambda i: (i, 0)),
            pl.BlockSpec((PK * 128, 128), lambda i: (0, 0)),
        ],
        out_specs=[
            pl.BlockSpec((BRP, 128), lambda i: (i, 0)),
            pl.BlockSpec((BRP, 128), lambda i: (i, 0)),
        ],
        out_shape=[
            jax.ShapeDtypeStruct((NPK, 128), jnp.float32),
            jax.ShapeDtypeStruct((NPK, 128), jnp.float32),
        ],
    )(deg_p, xp, Wbd1)


def _tc_stage_mid(part_ref, xws_ref, dinv_ref, b_ref, w_ref, out_ref):
    dinv = dinv_ref[...]
    tot = part_ref[0] + part_ref[1] + xws_ref[...]
    h = dinv * tot + b_ref[...]
    h = jnp.where(h > 0, h, jnp.exp(h) - 1.0)
    hw = jnp.dot(h, w_ref[...],
                 preferred_element_type=jnp.float32,
                 precision=lax.Precision.HIGHEST)
    out_ref[...] = hw * dinv


def _stage_mid(part_p, xws_p, dinv_p, b_p, Wbd):
    grid = (NPK // BRP,)
    return pl.pallas_call(
        _tc_stage_mid,
        grid=grid,
        in_specs=[
            pl.BlockSpec((NC, BRP, 128), lambda i: (0, i, 0)),
            pl.BlockSpec((BRP, 128), lambda i: (i, 0)),
            pl.BlockSpec((BRP, 128), lambda i: (i, 0)),
            pl.BlockSpec((1, 128), lambda i: (0, 0)),
            pl.BlockSpec((128, 128), lambda i: (0, 0)),
        ],
        out_specs=pl.BlockSpec((BRP, 128), lambda i: (i, 0)),
        out_shape=jax.ShapeDtypeStruct((NPK, 128), jnp.float32),
    )(part_p, xws_p, dinv_p, b_p, Wbd)


def _tc_stage_out(part_ref, xws_ref, dinv_ref, b_ref, out_ref):
    tot = part_ref[0] + part_ref[1] + xws_ref[...]
    out_ref[...] = dinv_ref[...] * tot + b_ref[...]


def _stage_out(part_p, xws_p, dinv_p, b_p):
    grid = (NPK // BRP,)
    return pl.pallas_call(
        _tc_stage_out,
        grid=grid,
        in_specs=[
            pl.BlockSpec((NC, BRP, 128), lambda i: (0, i, 0)),
            pl.BlockSpec((BRP, 128), lambda i: (i, 0)),
            pl.BlockSpec((BRP, 128), lambda i: (i, 0)),
            pl.BlockSpec((1, 128), lambda i: (0, 0)),
        ],
        out_specs=pl.BlockSpec((BRP, 128), lambda i: (i, 0)),
        out_shape=jax.ShapeDtypeStruct((NPK, 128), jnp.float32),
    )(part_p, xws_p, dinv_p, b_p)


def kernel(x, edge_index, W1, b1, W2, b2, W3, b3):
    src = edge_index[0].astype(jnp.int32)
    dst = edge_index[1].astype(jnp.int32)
    npad_e = E_PAD - N_EDGES
    pad_src = jnp.arange(npad_e, dtype=jnp.int32) % N_NODES
    pad_dst = N_NODES + (jnp.arange(npad_e, dtype=jnp.int32) % (NPAD - N_NODES))
    src3 = jnp.concatenate([src, pad_src]).reshape(NW, NCHUNK, CHUNK)
    dst3 = jnp.concatenate([dst, pad_dst]).reshape(NW, NCHUNK, CHUNK)

    zerosD = jnp.zeros((NPAD, D), jnp.float32)
    zeros_deg = jnp.zeros((NPAD, DEG_W), jnp.float32)
    ones_h = jnp.ones((CHUNK, DEG_W), jnp.float32)

    eye4 = jnp.eye(PK, dtype=jnp.float32)
    Wbd1 = jnp.kron(eye4, W1)
    Wbd2 = jnp.kron(eye4, W2)
    W3p = jnp.pad(W3, ((0, 0), (0, D - W3.shape[1])))
    Wbd3 = jnp.kron(eye4, W3p)
    b1_p = jnp.tile(b1, PK).reshape(1, 128)
    b2_p = jnp.tile(b2, PK).reshape(1, 128)
    b3_p = jnp.tile(jnp.pad(b3, (0, D - b3.shape[0])), PK).reshape(1, 128)

    xp = jnp.zeros((NPAD, x.shape[1]), jnp.float32).at[:N_NODES].set(x)
    xp = xp.reshape(NPK, PK * x.shape[1])

    deg_p = _degree(dst3, ones_h, zeros_deg)
    dinv_p, xws1_p = _stage1(deg_p, xp, Wbd1)

    part1 = _aggregate(src3, dst3, xws1_p.reshape(NPAD, D), zerosD)
    xws2_p = _stage_mid(part1.reshape(NC, NPK, 128), xws1_p, dinv_p, b1_p,
                        Wbd2)

    part2 = _aggregate(src3, dst3, xws2_p.reshape(NPAD, D), zerosD)
    xws3_p = _stage_mid(part2.reshape(NC, NPK, 128), xws2_p, dinv_p, b2_p,
                        Wbd3)

    part3 = _aggregate(src3, dst3, xws3_p.reshape(NPAD, D), zerosD)
    out_p = _stage_out(part3.reshape(NC, NPK, 128), xws3_p, dinv_p, b3_p)

    return out_p.reshape(NPAD, D)[:N_NODES, :b3.shape[0]]


# speedup vs baseline: 65.8549x; 1.0961x over previous
import jax
import jax.numpy as jnp
import numpy as np
from jax import lax
from jax.experimental import pallas as pl
from jax.experimental.pallas import tpu as pltpu
from jax.experimental.pallas import tpu_sc as plsc

N_NODES = 10000
NPAD = 10240
N_EDGES = 320000
D = 32
PK = 128 // D
NPK = NPAD // PK

NC = 2
NS = 16
NW = NC * NS
E_PAD = 327680
EDGES_PER_TILE = E_PAD // NW
CHUNK = 128
NCHUNK = EDGES_PER_TILE // CHUNK
ROWS_PER_TILE = NPAD // NS
PROWS_PER_TILE = ROWS_PER_TILE // PK
DEG_W = 8
NBUF = 6
LOOKAHEAD = 3
_NPAD_E = E_PAD - N_EDGES
_PAD_BLOCK = np.stack([
    np.arange(_NPAD_E, dtype=np.int32) % N_NODES,
    N_NODES + (np.arange(_NPAD_E, dtype=np.int32) % (NPAD - N_NODES)),
])

_mesh = plsc.VectorSubcoreMesh(core_axis_name="c", subcore_axis_name="s",
                               num_cores=NC, num_subcores=NS)
_sc_params = pltpu.CompilerParams(use_tc_tiling_on_sc=False,
                                  needs_layout_passes=False)


def _wid(c, s):
    return s * NC + c


def _sc_degree(e3, ones_h, zeros_h, deg_p,
               idx_v, ones_v, slab_v, pack_v, acc_sh, sem):
    c = lax.axis_index("c")
    s = lax.axis_index("s")
    w = _wid(c, s)
    pltpu.sync_copy(zeros_h.at[pl.ds(s * ROWS_PER_TILE, ROWS_PER_TILE)],
                    acc_sh.at[pl.ds(s * ROWS_PER_TILE, ROWS_PER_TILE)])
    pltpu.sync_copy(ones_h, ones_v)
    pltpu.sync_copy(e3.at[1, w], idx_v)
    plsc.subcore_barrier()

    def body(j, carry):
        @pl.when(j >= NBUF)
        def _():
            pltpu.make_async_copy(ones_v, acc_sh.at[idx_v.at[0]],
                                  sem.at[j % NBUF]).wait()
        pltpu.async_copy(ones_v, acc_sh.at[idx_v.at[j]], sem.at[j % NBUF],
                         add=True)
        return carry

    lax.fori_loop(0, NCHUNK, body, 0)
    for b in range(NBUF):
        pltpu.make_async_copy(ones_v, acc_sh.at[idx_v.at[0]],
                              sem.at[b]).wait()
    plsc.subcore_barrier()

    pltpu.sync_copy(acc_sh.at[pl.ds(s * ROWS_PER_TILE, ROWS_PER_TILE)],
                    slab_v)

    def pack_row(i, carry):
        for half in range(8):
            node = PK * i + half // 2
            idx = jnp.full((16,), node, jnp.int32)
            col = jnp.zeros((16,), jnp.int32)
            v = plsc.load_gather(slab_v, [idx, col])
            pack_v[i, pl.ds(16 * half, 16)] = v
        return carry

    lax.fori_loop(0, PROWS_PER_TILE, pack_row, 0)
    pltpu.sync_copy(pack_v,
                    deg_p.at[c, pl.ds(s * PROWS_PER_TILE, PROWS_PER_TILE)])


def _degree(e3, ones_h, zeros_h):
    f = pl.kernel(
        _sc_degree,
        out_type=jax.ShapeDtypeStruct((NC, NPK, 128), jnp.float32),
        mesh=_mesh,
        compiler_params=_sc_params,
        scratch_types=[
            pltpu.VMEM((NCHUNK, CHUNK), jnp.int32),
            pltpu.VMEM((CHUNK, DEG_W), jnp.float32),
            pltpu.VMEM((ROWS_PER_TILE, DEG_W), jnp.float32),
            pltpu.VMEM((PROWS_PER_TILE, 128), jnp.float32),
            pltpu.VMEM_SHARED((NPAD, DEG_W), jnp.float32),
            pltpu.SemaphoreType.DMA((NBUF,)),
        ],
    )
    return f(e3, ones_h, zeros_h)


def _sc_aggregate(e3, xws, zeros_h, part,
                  sidx_v, didx_v, rows_v, acc_sh, gsem, ssem):
    c = lax.axis_index("c")
    s = lax.axis_index("s")
    w = _wid(c, s)
    pltpu.sync_copy(zeros_h.at[pl.ds(s * ROWS_PER_TILE, ROWS_PER_TILE)],
                    acc_sh.at[pl.ds(s * ROWS_PER_TILE, ROWS_PER_TILE)])
    pltpu.sync_copy(e3.at[0, w], sidx_v)
    pltpu.sync_copy(e3.at[1, w], didx_v)
    plsc.subcore_barrier()

    def gather(j, b):
        pltpu.async_copy(xws.at[sidx_v.at[j]], rows_v.at[b], gsem.at[b])

    def scatter(j, b):
        pltpu.async_copy(rows_v.at[b], acc_sh.at[didx_v.at[j]], ssem.at[b],
                         add=True)

    for j in range(LOOKAHEAD):
        gather(j, j % NBUF)

    def body(j, carry):
        bg = (j + LOOKAHEAD) % NBUF

        @pl.when(j + LOOKAHEAD < NCHUNK)
        def _():
            @pl.when(j + LOOKAHEAD >= NBUF)
            def _():
                pltpu.make_async_copy(rows_v.at[bg],
                                      acc_sh.at[didx_v.at[0]],
                                      ssem.at[bg]).wait()
            gather(j + LOOKAHEAD, bg)

        b = j % NBUF
        pltpu.make_async_copy(xws.at[sidx_v.at[j]], rows_v.at[b],
                              gsem.at[b]).wait()
        scatter(j, b)
        return carry

    lax.fori_loop(0, NCHUNK, body, 0)
    for b in range(NBUF):
        pltpu.make_async_copy(rows_v.at[b], acc_sh.at[didx_v.at[0]],
                              ssem.at[b]).wait()
    plsc.subcore_barrier()
    pltpu.sync_copy(acc_sh.at[pl.ds(s * ROWS_PER_TILE, ROWS_PER_TILE)],
                    part.at[c, pl.ds(s * ROWS_PER_TILE, ROWS_PER_TILE)])


def _aggregate(e3, xws, zeros_h):
    f = pl.kernel(
        _sc_aggregate,
        out_type=jax.ShapeDtypeStruct((NC, NPAD, D), jnp.float32),
        mesh=_mesh,
        compiler_params=_sc_params,
        scratch_types=[
            pltpu.VMEM((NCHUNK, CHUNK), jnp.int32),
            pltpu.VMEM((NCHUNK, CHUNK), jnp.int32),
            pltpu.VMEM((NBUF, CHUNK, D), jnp.float32),
            pltpu.VMEM_SHARED((NPAD, D), jnp.float32),
            pltpu.SemaphoreType.DMA((NBUF,)),
            pltpu.SemaphoreType.DMA((NBUF,)),
        ],
    )
    return f(e3, xws, zeros_h)


BRP = 320


def _tc_stage1(deg_ref, xp_ref, w_ref, dinv_ref, xws_ref):
    dinv = lax.rsqrt(deg_ref[0] + deg_ref[1] + 1.0)
    dinv_ref[...] = dinv
    xw = jnp.dot(xp_ref[...], w_ref[...],
                 preferred_element_type=jnp.float32,
                 precision=lax.Precision.HIGHEST)
    xws_ref[...] = xw * dinv


def _stage1(deg_p, xp, Wbd1):
    grid = (NPK // BRP,)
    return pl.pallas_call(
        _tc_stage1,
        grid=grid,
        in_specs=[
            pl.BlockSpec((NC, BRP, 128), lambda i: (0, i, 0)),
            pl.BlockSpec((BRP, PK * 128), lambda i: (i, 0)),
            pl.BlockSpec((PK * 128, 128), lambda i: (0, 0)),
        ],
        out_specs=[
            pl.BlockSpec((BRP, 128), lambda i: (i, 0)),
            pl.BlockSpec((BRP, 128), lambda i: (i, 0)),
        ],
        out_shape=[
            jax.ShapeDtypeStruct((NPK, 128), jnp.float32),
            jax.ShapeDtypeStruct((NPK, 128), jnp.float32),
        ],
    )(deg_p, xp, Wbd1)


def _tc_stage_mid(part_ref, xws_ref, dinv_ref, b_ref, w_ref, out_ref):
    dinv = dinv_ref[...]
    tot = part_ref[0] + part_ref[1] + xws_ref[...]
    h = dinv * tot + b_ref[...]
    h = jnp.where(h > 0, h, jnp.exp(h) - 1.0)
    hw = jnp.dot(h, w_ref[...],
                 preferred_element_type=jnp.float32,
                 precision=lax.Precision.HIGHEST)
    out_ref[...] = hw * dinv


def _stage_mid(part_p, xws_p, dinv_p, b_p, Wbd):
    grid = (NPK // BRP,)
    return pl.pallas_call(
        _tc_stage_mid,
        grid=grid,
        in_specs=[
            pl.BlockSpec((NC, BRP, 128), lambda i: (0, i, 0)),
            pl.BlockSpec((BRP, 128), lambda i: (i, 0)),
            pl.BlockSpec((BRP, 128), lambda i: (i, 0)),
            pl.BlockSpec((1, 128), lambda i: (0, 0)),
            pl.BlockSpec((128, 128), lambda i: (0, 0)),
        ],
        out_specs=pl.BlockSpec((BRP, 128), lambda i: (i, 0)),
        out_shape=jax.ShapeDtypeStruct((NPK, 128), jnp.float32),
    )(part_p, xws_p, dinv_p, b_p, Wbd)


def _tc_stage_out(part_ref, xws_ref, dinv_ref, b_ref, out_ref):
    tot = part_ref[0] + part_ref[1] + xws_ref[...]
    out_ref[...] = dinv_ref[...] * tot + b_ref[...]


def _stage_out(part_p, xws_p, dinv_p, b_p):
    grid = (NPK // BRP,)
    return pl.pallas_call(
        _tc_stage_out,
        grid=grid,
        in_specs=[
            pl.BlockSpec((NC, BRP, 128), lambda i: (0, i, 0)),
            pl.BlockSpec((BRP, 128), lambda i: (i, 0)),
            pl.BlockSpec((BRP, 128), lambda i: (i, 0)),
            pl.BlockSpec((1, 128), lambda i: (0, 0)),
        ],
        out_specs=pl.BlockSpec((BRP, 128), lambda i: (i, 0)),
        out_shape=jax.ShapeDtypeStruct((NPK, 128), jnp.float32),
    )(part_p, xws_p, dinv_p, b_p)


def kernel(x, edge_index, W1, b1, W2, b2, W3, b3):
    e3 = jnp.concatenate(
        [edge_index.astype(jnp.int32), jnp.asarray(_PAD_BLOCK)], axis=1
    ).reshape(2, NW, NCHUNK, CHUNK)

    zerosD = jnp.zeros((NPAD, D), jnp.float32)
    zeros_deg = jnp.zeros((NPAD, DEG_W), jnp.float32)
    ones_h = jnp.ones((CHUNK, DEG_W), jnp.float32)

    eye4 = jnp.eye(PK, dtype=jnp.float32)
    Wbd1 = jnp.kron(eye4, W1)
    Wbd2 = jnp.kron(eye4, W2)
    W3p = jnp.pad(W3, ((0, 0), (0, D - W3.shape[1])))
    Wbd3 = jnp.kron(eye4, W3p)
    b1_p = jnp.tile(b1, PK).reshape(1, 128)
    b2_p = jnp.tile(b2, PK).reshape(1, 128)
    b3_p = jnp.tile(jnp.pad(b3, (0, D - b3.shape[0])), PK).reshape(1, 128)

    xp = jnp.zeros((NPAD, x.shape[1]), jnp.float32).at[:N_NODES].set(x)
    xp = xp.reshape(NPK, PK * x.shape[1])

    deg_p = _degree(e3, ones_h, zeros_deg)
    dinv_p, xws1_p = _stage1(deg_p, xp, Wbd1)

    part1 = _aggregate(e3, xws1_p.reshape(NPAD, D), zerosD)
    xws2_p = _stage_mid(part1.reshape(NC, NPK, 128), xws1_p, dinv_p, b1_p,
                        Wbd2)

    part2 = _aggregate(e3, xws2_p.reshape(NPAD, D), zerosD)
    xws3_p = _stage_mid(part2.reshape(NC, NPK, 128), xws2_p, dinv_p, b2_p,
                        Wbd3)

    part3 = _aggregate(e3, xws3_p.reshape(NPAD, D), zerosD)
    out_p = _stage_out(part3.reshape(NC, NPK, 128), xws3_p, dinv_p, b3_p)

    return out_p.reshape(NPAD, D)[:N_NODES, :b3.shape[0]]


# speedup vs baseline: 68.2247x; 1.0360x over previous
import jax
import jax.numpy as jnp
import numpy as np
from jax import lax
from jax.experimental import pallas as pl
from jax.experimental.pallas import tpu as pltpu
from jax.experimental.pallas import tpu_sc as plsc

N_NODES = 10000
NPAD = 10240
N_EDGES = 320000
D = 32
PK = 128 // D
NPK = NPAD // PK

NC = 2
NS = 16
NW = NC * NS
E_PAD = 327680
EDGES_PER_TILE = E_PAD // NW
CHUNK = 128
NCHUNK = EDGES_PER_TILE // CHUNK
ROWS_PER_TILE = NPAD // NS
PROWS_PER_TILE = ROWS_PER_TILE // PK
DEG_W = 8
NBUF = 6
LOOKAHEAD = 3
_NPAD_E = E_PAD - N_EDGES
_PAD_BLOCK = np.stack([
    np.arange(_NPAD_E, dtype=np.int32) % N_NODES,
    N_NODES + (np.arange(_NPAD_E, dtype=np.int32) % (NPAD - N_NODES)),
])

_T_EXPAND = np.zeros((128, 512), np.float32)
for _m in range(4):
    for _c in range(128):
        _T_EXPAND[8 * (4 * _m + _c // 32) + (_c % 8), 128 * _m + _c] = 1.0
_T_EXPAND = jnp.asarray(_T_EXPAND)
_T_SEL = np.zeros((128, 64), np.float32)
for _m in range(4):
    for _c in range(16):
        _T_SEL[32 * _m + _c, 16 * _m + _c] = 1.0
_T_SEL = jnp.asarray(_T_SEL)

_mesh = plsc.VectorSubcoreMesh(core_axis_name="c", subcore_axis_name="s",
                               num_cores=NC, num_subcores=NS)
_sc_params = pltpu.CompilerParams(use_tc_tiling_on_sc=False,
                                  needs_layout_passes=False)


def _wid(c, s):
    return s * NC + c


def _sc_degree(e3, ones_h, zeros_h, deg_part, idx_v, ones_v, acc_sh, sem):
    c = lax.axis_index("c")
    s = lax.axis_index("s")
    w = _wid(c, s)
    pltpu.sync_copy(zeros_h.at[pl.ds(s * ROWS_PER_TILE, ROWS_PER_TILE)],
                    acc_sh.at[pl.ds(s * ROWS_PER_TILE, ROWS_PER_TILE)])
    pltpu.sync_copy(ones_h, ones_v)
    pltpu.sync_copy(e3.at[1, w], idx_v)
    plsc.subcore_barrier()

    def body(j, carry):
        @pl.when(j >= NBUF)
        def _():
            pltpu.make_async_copy(ones_v, acc_sh.at[idx_v.at[0]],
                                  sem.at[j % NBUF]).wait()
        pltpu.async_copy(ones_v, acc_sh.at[idx_v.at[j]], sem.at[j % NBUF],
                         add=True)
        return carry

    lax.fori_loop(0, NCHUNK, body, 0)
    for b in range(NBUF):
        pltpu.make_async_copy(ones_v, acc_sh.at[idx_v.at[0]],
                              sem.at[b]).wait()
    plsc.subcore_barrier()
    pltpu.sync_copy(acc_sh.at[pl.ds(s * ROWS_PER_TILE, ROWS_PER_TILE)],
                    deg_part.at[c, pl.ds(s * ROWS_PER_TILE, ROWS_PER_TILE)])


def _degree(e3, ones_h, zeros_h):
    f = pl.kernel(
        _sc_degree,
        out_type=jax.ShapeDtypeStruct((NC, NPAD, DEG_W), jnp.float32),
        mesh=_mesh,
        compiler_params=_sc_params,
        scratch_types=[
            pltpu.VMEM((NCHUNK, CHUNK), jnp.int32),
            pltpu.VMEM((CHUNK, DEG_W), jnp.float32),
            pltpu.VMEM_SHARED((NPAD, DEG_W), jnp.float32),
            pltpu.SemaphoreType.DMA((NBUF,)),
        ],
    )
    return f(e3, ones_h, zeros_h)


def _sc_aggregate(e3, xws, zeros_h, part,
                  sidx_v, didx_v, rows_v, acc_sh, gsem, ssem):
    c = lax.axis_index("c")
    s = lax.axis_index("s")
    w = _wid(c, s)
    pltpu.sync_copy(zeros_h.at[pl.ds(s * ROWS_PER_TILE, ROWS_PER_TILE)],
                    acc_sh.at[pl.ds(s * ROWS_PER_TILE, ROWS_PER_TILE)])
    pltpu.sync_copy(e3.at[0, w], sidx_v)
    pltpu.sync_copy(e3.at[1, w], didx_v)
    plsc.subcore_barrier()

    def gather(j, b):
        pltpu.async_copy(xws.at[sidx_v.at[j]], rows_v.at[b], gsem.at[b])

    def scatter(j, b):
        pltpu.async_copy(rows_v.at[b], acc_sh.at[didx_v.at[j]], ssem.at[b],
                         add=True)

    for j in range(LOOKAHEAD):
        gather(j, j % NBUF)

    def body(j, carry):
        bg = (j + LOOKAHEAD) % NBUF

        @pl.when(j + LOOKAHEAD < NCHUNK)
        def _():
            @pl.when(j + LOOKAHEAD >= NBUF)
            def _():
                pltpu.make_async_copy(rows_v.at[bg],
                                      acc_sh.at[didx_v.at[0]],
                                      ssem.at[bg]).wait()
            gather(j + LOOKAHEAD, bg)

        b = j % NBUF
        pltpu.make_async_copy(xws.at[sidx_v.at[j]], rows_v.at[b],
                              gsem.at[b]).wait()
        scatter(j, b)
        return carry

    lax.fori_loop(0, NCHUNK, body, 0)
    for b in range(NBUF):
        pltpu.make_async_copy(rows_v.at[b], acc_sh.at[didx_v.at[0]],
                              ssem.at[b]).wait()
    plsc.subcore_barrier()
    pltpu.sync_copy(acc_sh.at[pl.ds(s * ROWS_PER_TILE, ROWS_PER_TILE)],
                    part.at[c, pl.ds(s * ROWS_PER_TILE, ROWS_PER_TILE)])


def _aggregate(e3, xws, zeros_h):
    f = pl.kernel(
        _sc_aggregate,
        out_type=jax.ShapeDtypeStruct((NC, NPAD, D), jnp.float32),
        mesh=_mesh,
        compiler_params=_sc_params,
        scratch_types=[
            pltpu.VMEM((NCHUNK, CHUNK), jnp.int32),
            pltpu.VMEM((NCHUNK, CHUNK), jnp.int32),
            pltpu.VMEM((NBUF, CHUNK, D), jnp.float32),
            pltpu.VMEM_SHARED((NPAD, D), jnp.float32),
            pltpu.SemaphoreType.DMA((NBUF,)),
            pltpu.SemaphoreType.DMA((NBUF,)),
        ],
    )
    return f(e3, xws, zeros_h)


BRP = 320


def _tc_stage1(deg_ref, xp_ref, w_ref, t_ref, dinv_ref, xws_ref):
    dinv8 = lax.rsqrt(deg_ref[0] + deg_ref[1] + 1.0)
    dinv4 = jnp.dot(dinv8, t_ref[...],
                    preferred_element_type=jnp.float32,
                    precision=lax.Precision.HIGHEST)
    dinv = dinv4.reshape(dinv4.shape[0] * PK, 128)
    dinv_ref[...] = dinv
    xw = jnp.dot(xp_ref[...], w_ref[...],
                 preferred_element_type=jnp.float32,
                 precision=lax.Precision.HIGHEST)
    xws_ref[...] = xw * dinv


def _stage1(deg8, xp, Wbd1):
    grid = (NPK // BRP,)
    return pl.pallas_call(
        _tc_stage1,
        grid=grid,
        in_specs=[
            pl.BlockSpec((NC, BRP // PK, 128), lambda i: (0, i, 0)),
            pl.BlockSpec((BRP, PK * 128), lambda i: (i, 0)),
            pl.BlockSpec((PK * 128, 128), lambda i: (0, 0)),
            pl.BlockSpec((128, PK * 128), lambda i: (0, 0)),
        ],
        out_specs=[
            pl.BlockSpec((BRP, 128), lambda i: (i, 0)),
            pl.BlockSpec((BRP, 128), lambda i: (i, 0)),
        ],
        out_shape=[
            jax.ShapeDtypeStruct((NPK, 128), jnp.float32),
            jax.ShapeDtypeStruct((NPK, 128), jnp.float32),
        ],
    )(deg8, xp, Wbd1, _T_EXPAND)


def _tc_stage_mid(part_ref, xws_ref, dinv_ref, b_ref, w_ref, out_ref):
    dinv = dinv_ref[...]
    tot = part_ref[0] + part_ref[1] + xws_ref[...]
    h = dinv * tot + b_ref[...]
    h = jnp.where(h > 0, h, jnp.exp(h) - 1.0)
    hw = jnp.dot(h, w_ref[...],
                 preferred_element_type=jnp.float32,
                 precision=lax.Precision.HIGHEST)
    out_ref[...] = hw * dinv


def _stage_mid(part_p, xws_p, dinv_p, b_p, Wbd):
    grid = (NPK // BRP,)
    return pl.pallas_call(
        _tc_stage_mid,
        grid=grid,
        in_specs=[
            pl.BlockSpec((NC, BRP, 128), lambda i: (0, i, 0)),
            pl.BlockSpec((BRP, 128), lambda i: (i, 0)),
            pl.BlockSpec((BRP, 128), lambda i: (i, 0)),
            pl.BlockSpec((1, 128), lambda i: (0, 0)),
            pl.BlockSpec((128, 128), lambda i: (0, 0)),
        ],
        out_specs=pl.BlockSpec((BRP, 128), lambda i: (i, 0)),
        out_shape=jax.ShapeDtypeStruct((NPK, 128), jnp.float32),
    )(part_p, xws_p, dinv_p, b_p, Wbd)


def _tc_stage_out(part_ref, xws_ref, dinv_ref, b_ref, t_ref, out_ref):
    tot = part_ref[0] + part_ref[1] + xws_ref[...]
    full = dinv_ref[...] * tot + b_ref[...]
    out_ref[...] = jnp.dot(full, t_ref[...],
                           preferred_element_type=jnp.float32,
                           precision=lax.Precision.HIGHEST)


def _stage_out(part_p, xws_p, dinv_p, b_p):
    grid = (NPK // BRP,)
    return pl.pallas_call(
        _tc_stage_out,
        grid=grid,
        in_specs=[
            pl.BlockSpec((NC, BRP, 128), lambda i: (0, i, 0)),
            pl.BlockSpec((BRP, 128), lambda i: (i, 0)),
            pl.BlockSpec((BRP, 128), lambda i: (i, 0)),
            pl.BlockSpec((1, 128), lambda i: (0, 0)),
            pl.BlockSpec((128, 64), lambda i: (0, 0)),
        ],
        out_specs=pl.BlockSpec((BRP, 64), lambda i: (i, 0)),
        out_shape=jax.ShapeDtypeStruct((NPK, 64), jnp.float32),
    )(part_p, xws_p, dinv_p, b_p, _T_SEL)


def kernel(x, edge_index, W1, b1, W2, b2, W3, b3):
    e3 = jnp.concatenate(
        [edge_index.astype(jnp.int32), jnp.asarray(_PAD_BLOCK)], axis=1
    ).reshape(2, NW, NCHUNK, CHUNK)

    zerosD = jnp.zeros((NPAD, D), jnp.float32)
    zeros_deg = jnp.zeros((NPAD, DEG_W), jnp.float32)
    ones_h = jnp.ones((CHUNK, DEG_W), jnp.float32)

    eye4 = jnp.eye(PK, dtype=jnp.float32)
    Wbd1 = jnp.kron(eye4, W1)
    Wbd2 = jnp.kron(eye4, W2)
    W3p = jnp.pad(W3, ((0, 0), (0, D - W3.shape[1])))
    Wbd3 = jnp.kron(eye4, W3p)
    b1_p = jnp.tile(b1, PK).reshape(1, 128)
    b2_p = jnp.tile(b2, PK).reshape(1, 128)
    b3_p = jnp.tile(jnp.pad(b3, (0, D - b3.shape[0])), PK).reshape(1, 128)

    xp = jnp.zeros((NPAD, x.shape[1]), jnp.float32).at[:N_NODES].set(x)
    xp = xp.reshape(NPK, PK * x.shape[1])

    deg_part = _degree(e3, ones_h, zeros_deg)
    dinv_p, xws1_p = _stage1(deg_part.reshape(NC, NPAD // 16, 128), xp, Wbd1)

    part1 = _aggregate(e3, xws1_p.reshape(NPAD, D), zerosD)
    xws2_p = _stage_mid(part1.reshape(NC, NPK, 128), xws1_p, dinv_p, b1_p,
                        Wbd2)

    part2 = _aggregate(e3, xws2_p.reshape(NPAD, D), zerosD)
    xws3_p = _stage_mid(part2.reshape(NC, NPK, 128), xws2_p, dinv_p, b2_p,
                        Wbd3)

    part3 = _aggregate(e3, xws3_p.reshape(NPAD, D), zerosD)
    out_p = _stage_out(part3.reshape(NC, NPK, 128), xws3_p, dinv_p, b3_p)

    return out_p.reshape(NPAD, 16)[:N_NODES]
